# Initial kernel scaffold; baseline (speedup 1.0000x reference)
import jax
import jax.numpy as jnp
from jax.experimental import pallas as pl


def kernel(features, edge_ori, w0, w1):
    raise NotImplementedError("write your pallas kernel here")


# speedup vs baseline: 14.7013x; 14.7013x over previous
import jax
import jax.numpy as jnp
from jax.experimental import pallas as pl

_KK = 33
_BISECT_ITERS = 30


def _emb_body(f_ref, w0_ref, w1_ref, emb_ref):
    h = jnp.maximum(f_ref[:] * w0_ref[:], 0.0) * w1_ref[:]
    sq = jnp.sum(h * h, axis=1, keepdims=True)
    emb_ref[:] = h * jax.lax.rsqrt(jnp.maximum(sq, 1e-24))


def _topk_body(emb_blk_ref, embT_ref, out_ref):
    sim = jnp.dot(emb_blk_ref[:], embT_ref[:], preferred_element_type=jnp.float32)
    out_ref[:] = sim
    blk = sim.shape[0]

    def bisect(_, carry):
        lo, hi = carry
        mid = 0.5 * (lo + hi)
        cnt = jnp.sum((out_ref[:] >= mid).astype(jnp.float32), axis=1, keepdims=True)
        ge = cnt >= float(_KK)
        return jnp.where(ge, mid, lo), jnp.where(ge, hi, mid)

    lo0 = jnp.full((blk, 1), -1.05, jnp.float32)
    hi0 = jnp.full((blk, 1), 1.05, jnp.float32)
    lo, _ = jax.lax.fori_loop(0, _BISECT_ITERS, bisect, (lo0, hi0))

    sim = out_ref[:]
    out_ref[:] = jnp.where(sim >= lo, jnp.maximum(sim, 0.0), 0.0)


def kernel(features, edge_ori, w0, w1):
    del edge_ori
    n, d = features.shape
    emb = pl.pallas_call(
        _emb_body,
        out_shape=jax.ShapeDtypeStruct((n, d), jnp.float32),
    )(features, w0.reshape(1, d), w1.reshape(1, d))

    emb_t = emb.T

    blk = 200 if n % 200 == 0 else 8
    out = pl.pallas_call(
        _topk_body,
        grid=(n // blk,),
        in_specs=[
            pl.BlockSpec((blk, d), lambda i: (i, 0)),
            pl.BlockSpec((d, n), lambda i: (0, 0)),
        ],
        out_specs=pl.BlockSpec((blk, n), lambda i: (i, 0)),
        out_shape=jax.ShapeDtypeStruct((n, n), jnp.float32),
    )(emb, emb_t)
    return out


# speedup vs baseline: 16.8188x; 1.1440x over previous
import jax
import jax.numpy as jnp
from jax.experimental import pallas as pl
from jax.experimental.pallas import tpu as pltpu

_KK = 33
_BISECT_ITERS = 25


def _emb_body(f_ref, w0_ref, w1_ref, emb_ref):
    h = jnp.maximum(f_ref[:] * w0_ref[:], 0.0) * w1_ref[:]
    sq = jnp.sum(h * h, axis=1, keepdims=True)
    emb_ref[:] = h * jax.lax.rsqrt(jnp.maximum(sq, 1e-24))


def _topk_body(emb_blk_ref, embT_ref, out_ref):
    sim = jnp.dot(emb_blk_ref[:], embT_ref[:], preferred_element_type=jnp.float32)
    out_ref[:] = sim
    blk = sim.shape[0]

    def bisect(_, carry):
        lo, hi = carry
        mid = 0.5 * (lo + hi)
        cnt = jnp.count_nonzero(out_ref[:] >= mid, axis=1, keepdims=True)
        ge = cnt >= _KK
        return jnp.where(ge, mid, lo), jnp.where(ge, hi, mid)

    lo0 = jnp.full((blk, 1), -1.05, jnp.float32)
    hi0 = jnp.full((blk, 1), 1.05, jnp.float32)
    lo, _ = jax.lax.fori_loop(0, _BISECT_ITERS, bisect, (lo0, hi0))

    sim = out_ref[:]
    out_ref[:] = jnp.where(sim >= lo, jnp.maximum(sim, 0.0), 0.0)


def kernel(features, edge_ori, w0, w1):
    del edge_ori
    n, d = features.shape
    emb = pl.pallas_call(
        _emb_body,
        out_shape=jax.ShapeDtypeStruct((n, d), jnp.float32),
    )(features, w0.reshape(1, d), w1.reshape(1, d))

    emb_t = emb.T

    blk = 200 if n % 200 == 0 else 8
    out = pl.pallas_call(
        _topk_body,
        grid=(n // blk,),
        in_specs=[
            pl.BlockSpec((blk, d), lambda i: (i, 0)),
            pl.BlockSpec((d, n), lambda i: (0, 0)),
        ],
        out_specs=pl.BlockSpec((blk, n), lambda i: (i, 0)),
        out_shape=jax.ShapeDtypeStruct((n, n), jnp.float32),
        compiler_params=pltpu.CompilerParams(
            dimension_semantics=("parallel",),
        ),
    )(emb, emb_t)
    return out


# speedup vs baseline: 17.8059x; 1.0587x over previous
import jax
import jax.numpy as jnp
from jax.experimental import pallas as pl
from jax.experimental.pallas import tpu as pltpu

_KK = 33
_BISECT_ITERS = 25


def _emb_body(f_ref, w0_ref, w1_ref, emb_ref):
    h = jnp.maximum(f_ref[:] * w0_ref[:], 0.0) * w1_ref[:]
    sq = jnp.sum(h * h, axis=1, keepdims=True)
    emb_ref[:] = h * jax.lax.rsqrt(jnp.maximum(sq, 1e-24))


def _topk_body(emb_blk_ref, embT_ref, out_ref):
    sim = jnp.dot(emb_blk_ref[:], embT_ref[:], preferred_element_type=jnp.float32)
    out_ref[:] = sim
    blk = sim.shape[0]

    def bisect(_, carry):
        lo, hi = carry
        mid = 0.5 * (lo + hi)
        cnt = jnp.count_nonzero(out_ref[:] >= mid, axis=1, keepdims=True)
        ge = cnt >= _KK
        return jnp.where(ge, mid, lo), jnp.where(ge, hi, mid)

    lo0 = jnp.full((blk, 1), -1.05, jnp.float32)
    hi0 = jnp.full((blk, 1), 1.05, jnp.float32)
    lo, _ = jax.lax.fori_loop(0, _BISECT_ITERS, bisect, (lo0, hi0))

    sim = out_ref[:]
    out_ref[:] = jnp.where(sim >= lo, jnp.maximum(sim, 0.0), 0.0)


def kernel(features, edge_ori, w0, w1):
    del edge_ori
    n, d = features.shape
    emb = pl.pallas_call(
        _emb_body,
        out_shape=jax.ShapeDtypeStruct((n, d), jnp.float32),
    )(features, w0.reshape(1, d), w1.reshape(1, d))

    emb_t = emb.T

    blk = 400 if n % 400 == 0 else 8
    out = pl.pallas_call(
        _topk_body,
        grid=(n // blk,),
        in_specs=[
            pl.BlockSpec((blk, d), lambda i: (i, 0)),
            pl.BlockSpec((d, n), lambda i: (0, 0)),
        ],
        out_specs=pl.BlockSpec((blk, n), lambda i: (i, 0)),
        out_shape=jax.ShapeDtypeStruct((n, n), jnp.float32),
        compiler_params=pltpu.CompilerParams(
            dimension_semantics=("parallel",),
        ),
    )(emb, emb_t)
    return out
